# baseline (speedup 1.0000x reference)
import jax
import jax.numpy as jnp
from jax import lax
from jax.experimental import pallas as pl
from jax.experimental.pallas import tpu as pltpu
from jax.experimental.pallas import tpu_sc as plsc

B = 16384
D = 32
L = 16
NC = 2
NS = 16
NW = NC * NS
BPW = B // NW
CHUNK = 128
NCHUNK = BPW // CHUNK
GROUPS_PER_CHUNK = CHUNK // L


def _sc_body(uid_hbm, iid_hbm, ut_hbm, it_hbm, out_hbm,
             uid_v, iid_v, urows_v, irows_v, out_v, sem_u, sem_i):
    wid = lax.axis_index("s") * NC + lax.axis_index("c")
    base = wid * BPW

    pltpu.sync_copy(uid_hbm.at[wid], uid_v)
    pltpu.sync_copy(iid_hbm.at[wid], iid_v)

    copies = []
    for j in range(NCHUNK):
        copies.append(pltpu.async_copy(ut_hbm.at[uid_v.at[j]], urows_v.at[j], sem_u))
        copies.append(pltpu.async_copy(it_hbm.at[iid_v.at[j]], irows_v.at[j], sem_i))
    for c in copies:
        c.wait()

    for j in range(NCHUNK):
        jvec = jnp.full((L,), j, jnp.int32)

        def group(g, _, jvec=jvec, j=j):
            bvec = g * L + lax.iota(jnp.int32, L)
            acc = jnp.zeros((L,), jnp.float32)
            for d in range(D):
                dvec = jnp.full((L,), d, jnp.int32)
                uc = plsc.load_gather(urows_v, [jvec, bvec, dvec])
                vc = plsc.load_gather(irows_v, [jvec, bvec, dvec])
                acc = acc + uc * vc
            out_v[pl.ds(j * CHUNK + g * L, L)] = acc
            return 0

        lax.fori_loop(0, GROUPS_PER_CHUNK, group, 0)

    pltpu.sync_copy(out_v, out_hbm.at[pl.ds(base, BPW)])


def kernel(user_ids, item_ids, user_table, item_table):
    uid = user_ids.astype(jnp.int32).reshape(NW, NCHUNK, CHUNK)
    iid = item_ids.astype(jnp.int32).reshape(NW, NCHUNK, CHUNK)
    mesh = plsc.VectorSubcoreMesh(core_axis_name="c", subcore_axis_name="s")
    f = pl.kernel(
        _sc_body,
        mesh=mesh,
        compiler_params=pltpu.CompilerParams(
            use_tc_tiling_on_sc=False, needs_layout_passes=False),
        out_type=jax.ShapeDtypeStruct((B,), jnp.float32),
        scratch_types=[
            pltpu.VMEM((NCHUNK, CHUNK), jnp.int32),
            pltpu.VMEM((NCHUNK, CHUNK), jnp.int32),
            pltpu.VMEM((NCHUNK, CHUNK, D), jnp.float32),
            pltpu.VMEM((NCHUNK, CHUNK, D), jnp.float32),
            pltpu.VMEM((BPW,), jnp.float32),
            pltpu.SemaphoreType.DMA,
            pltpu.SemaphoreType.DMA,
        ],
    )
    return f(uid, iid, user_table, item_table)
